# baseline (speedup 1.0000x reference)
import dataclasses
import functools

import jax
import jax.numpy as jnp
from jax import lax
from jax.experimental import pallas as pl
from jax.experimental.pallas import tpu as pltpu
from jax.experimental.pallas import tpu_sc as plsc

N = 10000
NPAD = 10240
E = 330000
NTILES = 32
NCH = 84
T_EDGE = NCH * 128
E_PAD = NTILES * T_EDGE
P = 200000
NCH_D = 50
T_DEC = NCH_D * 128
P_PAD = NTILES * T_DEC

_MESH = plsc.VectorSubcoreMesh(core_axis_name="c", subcore_axis_name="s")

_SC_PARAMS = pltpu.CompilerParams()
if "needs_layout_passes" in pltpu.CompilerParams.__dataclass_fields__:
    _SC_PARAMS = dataclasses.replace(_SC_PARAMS, needs_layout_passes=False)
if "use_tc_tiling_on_sc" in pltpu.CompilerParams.__dataclass_fields__:
    _SC_PARAMS = dataclasses.replace(_SC_PARAMS, use_tc_tiling_on_sc=False)


def _tc_embed_body(x_ref, w_ref, av_ref, bv_ref, h_ref, as_ref, ad_ref, c_ref):
    h = jnp.dot(x_ref[...], w_ref[...], preferred_element_type=jnp.float32)
    h_ref[...] = h
    asrc = jnp.sum(h * av_ref[...][None, :], axis=1)
    adst = jnp.sum(h * bv_ref[...][None, :], axis=1)
    as_ref[...] = asrc
    ad_ref[...] = adst
    c_ref[...] = jnp.broadcast_to(jnp.max(asrc) + jnp.max(adst), (16,))


def _tc_embed(x, w, avec, bvec):
    m = x.shape[0]
    d = w.shape[1]
    return pl.pallas_call(
        _tc_embed_body,
        out_shape=[
            jax.ShapeDtypeStruct((m, d), jnp.float32),
            jax.ShapeDtypeStruct((m,), jnp.float32),
            jax.ShapeDtypeStruct((m,), jnp.float32),
            jax.ShapeDtypeStruct((16,), jnp.float32),
        ],
    )(x, w, avec, bvec)


def _tc_mid_body(p_ref, dp_ref, b_ref, w_ref, av_ref, bv_ref,
                 h_ref, as_ref, ad_ref, c_ref):
    den = dp_ref[0] + dp_ref[1] + 1e-16
    t = (p_ref[0] + p_ref[1]) / den[:, None] + b_ref[...][None, :]
    t = jnp.maximum(t, 0.0)
    h = jnp.dot(t, w_ref[...], preferred_element_type=jnp.float32)
    h_ref[...] = h
    asrc = jnp.sum(h * av_ref[...][None, :], axis=1)
    adst = jnp.sum(h * bv_ref[...][None, :], axis=1)
    as_ref[...] = asrc
    ad_ref[...] = adst
    c_ref[...] = jnp.broadcast_to(jnp.max(asrc) + jnp.max(adst), (16,))


def _tc_mid(parts, dp, b, w, avec, bvec):
    m = parts.shape[1]
    d = w.shape[1]
    return pl.pallas_call(
        _tc_mid_body,
        out_shape=[
            jax.ShapeDtypeStruct((m, d), jnp.float32),
            jax.ShapeDtypeStruct((m,), jnp.float32),
            jax.ShapeDtypeStruct((m,), jnp.float32),
            jax.ShapeDtypeStruct((16,), jnp.float32),
        ],
    )(parts, dp, b, w, avec, bvec)


def _tc_z_body(p_ref, dp_ref, b_ref, z_ref):
    den = dp_ref[0] + dp_ref[1] + 1e-16
    z_ref[...] = (p_ref[0] + p_ref[1]) / den[:, None] + b_ref[...][None, :]


def _tc_z(parts, dp, b):
    return pl.pallas_call(
        _tc_z_body,
        out_shape=jax.ShapeDtypeStruct(parts.shape[1:], jnp.float32),
    )(parts, dp, b)


def _tc_fold_body(a_ref, o_ref):
    lanes = lax.broadcasted_iota(jnp.int32, (128, 8), 0)
    cols = lax.broadcasted_iota(jnp.int32, (128, 8), 1)
    s = jnp.where(lanes // 16 == cols, 1.0, 0.0)
    o_ref[...] = jnp.dot(a_ref[...], s, preferred_element_type=jnp.float32)


def _tc_fold(acc):
    return pl.pallas_call(
        _tc_fold_body,
        out_shape=jax.ShapeDtypeStruct((acc.shape[0], 8), jnp.float32),
    )(acc)


def _sc_layer(h, asrc, adst, cvec, ei):
    m = asrc.shape[0]
    d = h.shape[1]
    ng = d // 16

    @functools.partial(
        pl.kernel,
        out_type=[
            jax.ShapeDtypeStruct((2, NPAD, d), jnp.float32),
            jax.ShapeDtypeStruct((2 * NPAD,), jnp.float32),
        ],
        mesh=_MESH,
        compiler_params=_SC_PARAMS,
        scratch_types=[
            pltpu.VMEM((NPAD,), jnp.float32),
            pltpu.VMEM((NPAD,), jnp.float32),
            pltpu.VMEM((16,), jnp.float32),
            pltpu.VMEM((4, 2, 128), jnp.int32),
            pltpu.VMEM((512,), jnp.float32),
            pltpu.VMEM((2, 128, d), jnp.float32),
            pltpu.VMEM((640,), jnp.float32),
            pltpu.VMEM((NPAD,), jnp.float32),
            pltpu.VMEM((16, 640), jnp.float32),
            pltpu.VMEM_SHARED((NPAD, d), jnp.float32),
            pltpu.VMEM_SHARED((16, NPAD), jnp.float32),
            pltpu.SemaphoreType.DMA,
            pltpu.SemaphoreType.DMA,
            pltpu.SemaphoreType.DMA,
            pltpu.SemaphoreType.DMA,
            pltpu.SemaphoreType.DMA,
            pltpu.SemaphoreType.DMA,
            pltpu.SemaphoreType.DMA,
            pltpu.SemaphoreType.DMA,
        ],
    )
    def scl(h_h, as_h, ad_h, c_h, ei_h, outp_h, dp_h,
            asrc_v, adst_v, c_v, sd_v, w_v, rows_v, zbuf,
            denom_v, stage_v, out_sh, dred_sh,
            ld0, ld1, ld2, ld3, g0, g1, sc0, sc1):
        cid = lax.axis_index("c")
        sid = lax.axis_index("s")
        wid = sid * 2 + cid
        base = wid * T_EDGE
        ld_sems = [ld0, ld1, ld2, ld3]
        g_sems = [g0, g1]
        sc_sems = [sc0, sc1]

        @pl.loop(0, 128)
        def _(r):
            for g_ in range(ng):
                rows_v[0, r, pl.ds(g_ * 16, 16)] = jnp.zeros((16,), jnp.float32)

        @pl.loop(0, NPAD // 16)
        def _(i):
            denom_v[pl.ds(i * 16, 16)] = jnp.zeros((16,), jnp.float32)

        @pl.loop(0, 5)
        def _(i):
            pltpu.sync_copy(rows_v.at[0],
                            out_sh.at[pl.ds((sid * 5 + i) * 128, 128)])

        pltpu.sync_copy(as_h.at[pl.ds(0, m)], asrc_v.at[pl.ds(0, m)])
        pltpu.sync_copy(ad_h.at[pl.ds(0, m)], adst_v.at[pl.ds(0, m)])
        pltpu.sync_copy(c_h, c_v)
        plsc.subcore_barrier()

        def load_start(b4, c):
            off = pl.multiple_of(
                jnp.minimum(base + c * 128, E_PAD - 128), 128)
            pltpu.make_async_copy(
                ei_h.at[:, pl.ds(off, 128)], sd_v.at[b4], ld_sems[b4]).start()

        def load_wait(b4):
            pltpu.make_async_copy(
                ei_h.at[:, pl.ds(0, 128)], sd_v.at[b4], ld_sems[b4]).wait()

        def gather_start(b4, b2):
            pltpu.make_async_copy(
                h_h.at[sd_v.at[b4, 0]], rows_v.at[b2], g_sems[b2]).start()

        def gather_wait(b4, b2):
            pltpu.make_async_copy(
                h_h.at[sd_v.at[b4, 0]], rows_v.at[b2], g_sems[b2]).wait()

        def scatter_start(b4, b2):
            pltpu.async_copy(rows_v.at[b2], out_sh.at[sd_v.at[b4, 1]],
                             sc_sems[b2], add=True)

        def scatter_wait(b4, b2):
            pltpu.make_async_copy(
                rows_v.at[b2], out_sh.at[sd_v.at[b4, 1]], sc_sems[b2]).wait()

        def compute_w(b4, c):
            cb = base + c * 128
            for k in range(8):
                sv = sd_v[b4, 0, pl.ds(k * 16, 16)]
                dv = sd_v[b4, 1, pl.ds(k * 16, 16)]
                av = plsc.load_gather(asrc_v, [sv])
                bv = plsc.load_gather(adst_v, [dv])
                e = av + bv
                e = jnp.where(e >= 0.0, e, 0.2 * e)
                wv = jnp.exp(e - c_v[...])
                gid = lax.iota(jnp.int32, 16) + (cb + k * 16)
                wv = jnp.where(gid < E, wv, 0.0)
                w_v[pl.ds(b4 * 128 + k * 16, 16)] = wv
                plsc.addupdate_scatter(denom_v, [dv], wv)

        def scale(b4, b2):
            @plsc.parallel_loop(0, 128, unroll=16)
            def _(r):
                widx = jnp.full((16,), b4 * 128, jnp.int32) + r
                wvec = plsc.load_gather(w_v, [widx])
                for g_ in range(ng):
                    rows_v[b2, r, pl.ds(g_ * 16, 16)] = (
                        rows_v[b2, r, pl.ds(g_ * 16, 16)] * wvec)

        def half(c, b4, b2, first):
            gather_wait(b4, b2)
            compute_w(b4, c)
            scale(b4, b2)
            scatter_start(b4, b2)
            load_start((b4 + 2) % 4, c + 2)
            nb4 = (b4 + 1) % 4
            nb2 = 1 - b2
            load_wait(nb4)
            if not first:
                scatter_wait((b4 + 3) % 4, nb2)
            gather_start(nb4, nb2)

        load_start(0, 0)
        load_start(1, 1)
        load_wait(0)
        gather_start(0, 0)
        half(0, 0, 0, True)
        half(1, 1, 1, False)
        half(2, 2, 0, False)
        half(3, 3, 1, False)

        @pl.loop(1, (NCH - 4) // 4 + 1)
        def _(gi):
            c0 = 4 * gi
            half(c0, 0, 0, False)
            half(c0 + 1, 1, 1, False)
            half(c0 + 2, 2, 0, False)
            half(c0 + 3, 3, 1, False)

        gather_wait(0, 0)
        load_wait(1)
        scatter_wait(3, 1)

        pltpu.sync_copy(denom_v, dred_sh.at[sid])
        plsc.subcore_barrier()

        @pl.loop(0, 16)
        def _(t):
            pltpu.sync_copy(dred_sh.at[t, pl.ds(sid * 640, 640)],
                            stage_v.at[t])

        @pl.loop(0, 40)
        def _(j):
            tot = stage_v[0, pl.ds(j * 16, 16)]
            for t in range(1, 16):
                tot = tot + stage_v[t, pl.ds(j * 16, 16)]
            zbuf[pl.ds(j * 16, 16)] = tot

        @pl.loop(0, 5)
        def _(i):
            s = (sid * 5 + i) * 128
            pltpu.sync_copy(out_sh.at[pl.ds(s, 128)],
                            outp_h.at[cid, pl.ds(s, 128)])

        pltpu.sync_copy(zbuf, dp_h.at[pl.ds(cid * NPAD + sid * 640, 640)])

    return scl(h, asrc, adst, cvec, ei)


def _sc_decode(z, eid):
    @functools.partial(
        pl.kernel,
        out_type=jax.ShapeDtypeStruct((P_PAD // 8, 128), jnp.float32),
        mesh=_MESH,
        compiler_params=_SC_PARAMS,
        scratch_types=[
            pltpu.VMEM((2, 2, 128), jnp.int32),
            pltpu.VMEM((2, 128, 32), jnp.float32),
            pltpu.VMEM((2, 128, 32), jnp.float32),
            pltpu.VMEM((2, 16, 128), jnp.float32),
            pltpu.SemaphoreType.DMA,
            pltpu.SemaphoreType.DMA,
            pltpu.SemaphoreType.DMA,
            pltpu.SemaphoreType.DMA,
            pltpu.SemaphoreType.DMA,
            pltpu.SemaphoreType.DMA,
        ],
    )
    def scd(z_h, ei_h, acc_h,
            idx_v, r0_v, r1_v, acc_v, ld0, ld1, g0, g1, st0, st1):
        cid = lax.axis_index("c")
        sid = lax.axis_index("s")
        wid = sid * 2 + cid
        base = wid * T_DEC
        ld_sems = [ld0, ld1]
        g_sems = [g0, g1]
        st_sems = [st0, st1]

        def load_start(b, c):
            off = pl.multiple_of(
                jnp.minimum(base + c * 128, P_PAD - 128), 128)
            pltpu.make_async_copy(
                ei_h.at[:, pl.ds(off, 128)], idx_v.at[b], ld_sems[b]).start()

        def load_wait(b):
            pltpu.make_async_copy(
                ei_h.at[:, pl.ds(0, 128)], idx_v.at[b], ld_sems[b]).wait()

        def gather_start(b):
            pltpu.make_async_copy(
                z_h.at[idx_v.at[b, 0]], r0_v.at[b], g_sems[b]).start()
            pltpu.make_async_copy(
                z_h.at[idx_v.at[b, 1]], r1_v.at[b], g_sems[b]).start()

        def gather_wait(b):
            pltpu.make_async_copy(
                z_h.at[idx_v.at[b, 0]], r0_v.at[b], g_sems[b]).wait()
            pltpu.make_async_copy(
                z_h.at[idx_v.at[b, 1]], r1_v.at[b], g_sems[b]).wait()

        def store_start(b, c):
            off = pl.multiple_of(base // 8 + c * 16, 16)
            pltpu.make_async_copy(
                acc_v.at[b], acc_h.at[pl.ds(off, 16)], st_sems[b]).start()

        def store_wait(b):
            pltpu.make_async_copy(
                acc_v.at[b], acc_h.at[pl.ds(0, 16)], st_sems[b]).wait()

        def half(c, b, first):
            gather_wait(b)
            if not first:
                store_wait(b)

            @plsc.parallel_loop(0, 128, unroll=16)
            def _(r):
                p0 = r0_v[b, r, pl.ds(0, 16)] * r1_v[b, r, pl.ds(0, 16)]
                p1 = r0_v[b, r, pl.ds(16, 16)] * r1_v[b, r, pl.ds(16, 16)]
                acc_v[b, r // 8, pl.ds((r % 8) * 16, 16)] = p0 + p1

            store_start(b, c)
            load_start(b, c + 2)
            nb = 1 - b
            load_wait(nb)
            gather_start(nb)

        load_start(0, 0)
        load_start(1, 1)
        load_wait(0)
        gather_start(0)
        half(0, 0, True)
        half(1, 1, True)

        @pl.loop(1, NCH_D // 2)
        def _(gi):
            half(2 * gi, 0, False)
            half(2 * gi + 1, 1, False)

        gather_wait(0)
        load_wait(1)
        store_wait(0)
        store_wait(1)

    return scd(z, eid)


def kernel(x, edge_index, pos_edge_index, neg_edge_index,
           W1, a1_src, a1_dst, b1, W2, a2_src, a2_dst, b2):
    loops = jnp.arange(N, dtype=edge_index.dtype)
    ei = jnp.concatenate(
        [edge_index, jnp.stack([loops, loops], axis=0)], axis=1)
    pad = E_PAD - E
    ei_pad = jnp.pad(ei, ((0, 0), (0, pad)))

    dec = jnp.concatenate([pos_edge_index, neg_edge_index], axis=1)
    dpad = P_PAD - P
    dec_pad = jnp.pad(dec, ((0, 0), (0, dpad)))

    h1, as1, ad1, c1 = _tc_embed(x, W1, a1_src, a1_dst)
    outp1, dp1 = _sc_layer(h1, as1, ad1, c1, ei_pad)

    h2, as2, ad2, c2 = _tc_mid(outp1, dp1.reshape(2, NPAD), b1,
                               W2, a2_src, a2_dst)
    outp2, dp2 = _sc_layer(h2, as2, ad2, c2, ei_pad)

    z = _tc_z(outp2, dp2.reshape(2, NPAD), b2)
    acc = _sc_decode(z, dec_pad)
    logits_pad = _tc_fold(acc)
    return logits_pad.reshape(P_PAD)[:P]


# speedup vs baseline: 1.0051x; 1.0051x over previous
import dataclasses
import functools

import jax
import jax.numpy as jnp
from jax import lax
from jax.experimental import pallas as pl
from jax.experimental.pallas import tpu as pltpu
from jax.experimental.pallas import tpu_sc as plsc

N = 10000
NPAD = 10240
E = 330000
NTILES = 32
NCH = 84
T_EDGE = NCH * 128
E_PAD = NTILES * T_EDGE
P = 200000
NCH_D = 50
T_DEC = NCH_D * 128
P_PAD = NTILES * T_DEC

_MESH = plsc.VectorSubcoreMesh(core_axis_name="c", subcore_axis_name="s")

_SC_PARAMS = pltpu.CompilerParams()
if "needs_layout_passes" in pltpu.CompilerParams.__dataclass_fields__:
    _SC_PARAMS = dataclasses.replace(_SC_PARAMS, needs_layout_passes=False)
if "use_tc_tiling_on_sc" in pltpu.CompilerParams.__dataclass_fields__:
    _SC_PARAMS = dataclasses.replace(_SC_PARAMS, use_tc_tiling_on_sc=False)


def _tc_embed_body(x_ref, w_ref, av_ref, bv_ref, h_ref, as_ref, ad_ref, c_ref):
    h = jnp.dot(x_ref[...], w_ref[...], preferred_element_type=jnp.float32)
    h_ref[...] = h
    asrc = jnp.sum(h * av_ref[...][None, :], axis=1)
    adst = jnp.sum(h * bv_ref[...][None, :], axis=1)
    as_ref[...] = asrc
    ad_ref[...] = adst
    c_ref[...] = jnp.broadcast_to(jnp.max(asrc) + jnp.max(adst), (16,))


def _tc_embed(x, w, avec, bvec):
    m = x.shape[0]
    d = w.shape[1]
    return pl.pallas_call(
        _tc_embed_body,
        out_shape=[
            jax.ShapeDtypeStruct((m, d), jnp.float32),
            jax.ShapeDtypeStruct((m,), jnp.float32),
            jax.ShapeDtypeStruct((m,), jnp.float32),
            jax.ShapeDtypeStruct((16,), jnp.float32),
        ],
    )(x, w, avec, bvec)


def _tc_mid_body(p_ref, dp_ref, b_ref, w_ref, av_ref, bv_ref,
                 h_ref, as_ref, ad_ref, c_ref):
    den = dp_ref[0] + dp_ref[1] + 1e-16
    t = (p_ref[0] + p_ref[1]) / den[:, None] + b_ref[...][None, :]
    t = jnp.maximum(t, 0.0)
    h = jnp.dot(t, w_ref[...], preferred_element_type=jnp.float32)
    h_ref[...] = h
    asrc = jnp.sum(h * av_ref[...][None, :], axis=1)
    adst = jnp.sum(h * bv_ref[...][None, :], axis=1)
    as_ref[...] = asrc
    ad_ref[...] = adst
    c_ref[...] = jnp.broadcast_to(jnp.max(asrc) + jnp.max(adst), (16,))


def _tc_mid(parts, dp, b, w, avec, bvec):
    m = parts.shape[1]
    d = w.shape[1]
    return pl.pallas_call(
        _tc_mid_body,
        out_shape=[
            jax.ShapeDtypeStruct((m, d), jnp.float32),
            jax.ShapeDtypeStruct((m,), jnp.float32),
            jax.ShapeDtypeStruct((m,), jnp.float32),
            jax.ShapeDtypeStruct((16,), jnp.float32),
        ],
    )(parts, dp, b, w, avec, bvec)


def _tc_z_body(p_ref, dp_ref, b_ref, z_ref):
    den = dp_ref[0] + dp_ref[1] + 1e-16
    z_ref[...] = (p_ref[0] + p_ref[1]) / den[:, None] + b_ref[...][None, :]


def _tc_z(parts, dp, b):
    return pl.pallas_call(
        _tc_z_body,
        out_shape=jax.ShapeDtypeStruct(parts.shape[1:], jnp.float32),
    )(parts, dp, b)


def _tc_fold_body(a_ref, o_ref):
    lanes = lax.broadcasted_iota(jnp.int32, (128, 8), 0)
    cols = lax.broadcasted_iota(jnp.int32, (128, 8), 1)
    s = jnp.where(lanes // 16 == cols, 1.0, 0.0)
    o_ref[...] = jnp.dot(a_ref[...], s, preferred_element_type=jnp.float32)


def _tc_fold(acc):
    return pl.pallas_call(
        _tc_fold_body,
        out_shape=jax.ShapeDtypeStruct((acc.shape[0], 8), jnp.float32),
    )(acc)


def _sc_layer(h, asrc, adst, cvec, ei):
    m = asrc.shape[0]
    d = h.shape[1]
    ng = d // 16

    @functools.partial(
        pl.kernel,
        out_type=[
            jax.ShapeDtypeStruct((2, NPAD, d), jnp.float32),
            jax.ShapeDtypeStruct((2 * NPAD,), jnp.float32),
        ],
        mesh=_MESH,
        compiler_params=_SC_PARAMS,
        scratch_types=[
            pltpu.VMEM((NPAD,), jnp.float32),
            pltpu.VMEM((NPAD,), jnp.float32),
            pltpu.VMEM((16,), jnp.float32),
            pltpu.VMEM((4, 2, 128), jnp.int32),
            pltpu.VMEM((512,), jnp.float32),
            pltpu.VMEM((2, 128, d), jnp.float32),
            pltpu.VMEM((640,), jnp.float32),
            pltpu.VMEM((NPAD,), jnp.float32),
            pltpu.VMEM((16, 640), jnp.float32),
            pltpu.VMEM_SHARED((NPAD, d), jnp.float32),
            pltpu.VMEM_SHARED((16, NPAD), jnp.float32),
            pltpu.SemaphoreType.DMA,
            pltpu.SemaphoreType.DMA,
            pltpu.SemaphoreType.DMA,
            pltpu.SemaphoreType.DMA,
            pltpu.SemaphoreType.DMA,
            pltpu.SemaphoreType.DMA,
            pltpu.SemaphoreType.DMA,
            pltpu.SemaphoreType.DMA,
        ],
    )
    def scl(h_h, as_h, ad_h, c_h, ei_h, outp_h, dp_h,
            asrc_v, adst_v, c_v, sd_v, w_v, rows_v, zbuf,
            denom_v, stage_v, out_sh, dred_sh,
            ld0, ld1, ld2, ld3, g0, g1, sc0, sc1):
        cid = lax.axis_index("c")
        sid = lax.axis_index("s")
        wid = sid * 2 + cid
        base = wid * T_EDGE
        ld_sems = [ld0, ld1, ld2, ld3]
        g_sems = [g0, g1]
        sc_sems = [sc0, sc1]

        @pl.loop(0, 128)
        def _(r):
            for g_ in range(ng):
                rows_v[0, r, pl.ds(g_ * 16, 16)] = jnp.zeros((16,), jnp.float32)

        @pl.loop(0, NPAD // 16)
        def _(i):
            denom_v[pl.ds(i * 16, 16)] = jnp.zeros((16,), jnp.float32)

        @pl.loop(0, 5)
        def _(i):
            pltpu.sync_copy(rows_v.at[0],
                            out_sh.at[pl.ds((sid * 5 + i) * 128, 128)])

        pltpu.sync_copy(as_h.at[pl.ds(0, m)], asrc_v.at[pl.ds(0, m)])
        pltpu.sync_copy(ad_h.at[pl.ds(0, m)], adst_v.at[pl.ds(0, m)])
        pltpu.sync_copy(c_h, c_v)
        plsc.subcore_barrier()

        def load_start(b4, c):
            off = pl.multiple_of(
                jnp.minimum(base + c * 128, E_PAD - 128), 128)
            pltpu.make_async_copy(
                ei_h.at[:, pl.ds(off, 128)], sd_v.at[b4], ld_sems[b4]).start()

        def load_wait(b4):
            pltpu.make_async_copy(
                ei_h.at[:, pl.ds(0, 128)], sd_v.at[b4], ld_sems[b4]).wait()

        def gather_start(b4, b2):
            pltpu.make_async_copy(
                h_h.at[sd_v.at[b4, 0]], rows_v.at[b2], g_sems[b2]).start()

        def gather_wait(b4, b2):
            pltpu.make_async_copy(
                h_h.at[sd_v.at[b4, 0]], rows_v.at[b2], g_sems[b2]).wait()

        def scatter_start(b4, b2):
            pltpu.async_copy(rows_v.at[b2], out_sh.at[sd_v.at[b4, 1]],
                             sc_sems[b2], add=True)

        def scatter_wait(b4, b2):
            pltpu.make_async_copy(
                rows_v.at[b2], out_sh.at[sd_v.at[b4, 1]], sc_sems[b2]).wait()

        def compute_w(b4, c):
            cb = base + c * 128
            for k in range(8):
                sv = sd_v[b4, 0, pl.ds(k * 16, 16)]
                dv = sd_v[b4, 1, pl.ds(k * 16, 16)]
                av = plsc.load_gather(asrc_v, [sv])
                bv = plsc.load_gather(adst_v, [dv])
                e = av + bv
                e = jnp.where(e >= 0.0, e, 0.2 * e)
                wv = jnp.exp(e - c_v[...])
                gid = lax.iota(jnp.int32, 16) + (cb + k * 16)
                wv = jnp.where(gid < E, wv, 0.0)
                w_v[pl.ds(b4 * 128 + k * 16, 16)] = wv
                plsc.addupdate_scatter(denom_v, [dv], wv)

        def scale(b4, b2):
            @plsc.parallel_loop(0, 128, unroll=8)
            def _(r):
                widx = jnp.full((16,), b4 * 128, jnp.int32) + r
                wvec = plsc.load_gather(w_v, [widx])
                for g_ in range(ng):
                    rows_v[b2, r, pl.ds(g_ * 16, 16)] = (
                        rows_v[b2, r, pl.ds(g_ * 16, 16)] * wvec)

        def half(c, b4, b2, first):
            gather_wait(b4, b2)
            compute_w(b4, c)
            scale(b4, b2)
            scatter_start(b4, b2)
            load_start((b4 + 2) % 4, c + 2)
            nb4 = (b4 + 1) % 4
            nb2 = 1 - b2
            load_wait(nb4)
            if not first:
                scatter_wait((b4 + 3) % 4, nb2)
            gather_start(nb4, nb2)

        load_start(0, 0)
        load_start(1, 1)
        load_wait(0)
        gather_start(0, 0)
        half(0, 0, 0, True)
        half(1, 1, 1, False)
        half(2, 2, 0, False)
        half(3, 3, 1, False)

        @pl.loop(1, (NCH - 4) // 4 + 1)
        def _(gi):
            c0 = 4 * gi
            half(c0, 0, 0, False)
            half(c0 + 1, 1, 1, False)
            half(c0 + 2, 2, 0, False)
            half(c0 + 3, 3, 1, False)

        gather_wait(0, 0)
        load_wait(1)
        scatter_wait(3, 1)

        pltpu.sync_copy(denom_v, dred_sh.at[sid])
        plsc.subcore_barrier()

        @pl.loop(0, 16)
        def _(t):
            pltpu.sync_copy(dred_sh.at[t, pl.ds(sid * 640, 640)],
                            stage_v.at[t])

        @pl.loop(0, 40)
        def _(j):
            tot = stage_v[0, pl.ds(j * 16, 16)]
            for t in range(1, 16):
                tot = tot + stage_v[t, pl.ds(j * 16, 16)]
            zbuf[pl.ds(j * 16, 16)] = tot

        @pl.loop(0, 5)
        def _(i):
            s = (sid * 5 + i) * 128
            pltpu.sync_copy(out_sh.at[pl.ds(s, 128)],
                            outp_h.at[cid, pl.ds(s, 128)])

        pltpu.sync_copy(zbuf, dp_h.at[pl.ds(cid * NPAD + sid * 640, 640)])

    return scl(h, asrc, adst, cvec, ei)


def _sc_decode(z, eid):
    @functools.partial(
        pl.kernel,
        out_type=jax.ShapeDtypeStruct((P_PAD // 8, 128), jnp.float32),
        mesh=_MESH,
        compiler_params=_SC_PARAMS,
        scratch_types=[
            pltpu.VMEM((2, 2, 128), jnp.int32),
            pltpu.VMEM((2, 128, 32), jnp.float32),
            pltpu.VMEM((2, 128, 32), jnp.float32),
            pltpu.VMEM((2, 16, 128), jnp.float32),
            pltpu.SemaphoreType.DMA,
            pltpu.SemaphoreType.DMA,
            pltpu.SemaphoreType.DMA,
            pltpu.SemaphoreType.DMA,
            pltpu.SemaphoreType.DMA,
            pltpu.SemaphoreType.DMA,
        ],
    )
    def scd(z_h, ei_h, acc_h,
            idx_v, r0_v, r1_v, acc_v, ld0, ld1, g0, g1, st0, st1):
        cid = lax.axis_index("c")
        sid = lax.axis_index("s")
        wid = sid * 2 + cid
        base = wid * T_DEC
        ld_sems = [ld0, ld1]
        g_sems = [g0, g1]
        st_sems = [st0, st1]

        def load_start(b, c):
            off = pl.multiple_of(
                jnp.minimum(base + c * 128, P_PAD - 128), 128)
            pltpu.make_async_copy(
                ei_h.at[:, pl.ds(off, 128)], idx_v.at[b], ld_sems[b]).start()

        def load_wait(b):
            pltpu.make_async_copy(
                ei_h.at[:, pl.ds(0, 128)], idx_v.at[b], ld_sems[b]).wait()

        def gather_start(b):
            pltpu.make_async_copy(
                z_h.at[idx_v.at[b, 0]], r0_v.at[b], g_sems[b]).start()
            pltpu.make_async_copy(
                z_h.at[idx_v.at[b, 1]], r1_v.at[b], g_sems[b]).start()

        def gather_wait(b):
            pltpu.make_async_copy(
                z_h.at[idx_v.at[b, 0]], r0_v.at[b], g_sems[b]).wait()
            pltpu.make_async_copy(
                z_h.at[idx_v.at[b, 1]], r1_v.at[b], g_sems[b]).wait()

        def store_start(b, c):
            off = pl.multiple_of(base // 8 + c * 16, 16)
            pltpu.make_async_copy(
                acc_v.at[b], acc_h.at[pl.ds(off, 16)], st_sems[b]).start()

        def store_wait(b):
            pltpu.make_async_copy(
                acc_v.at[b], acc_h.at[pl.ds(0, 16)], st_sems[b]).wait()

        def half(c, b, first):
            gather_wait(b)
            if not first:
                store_wait(b)

            @plsc.parallel_loop(0, 128, unroll=8)
            def _(r):
                p0 = r0_v[b, r, pl.ds(0, 16)] * r1_v[b, r, pl.ds(0, 16)]
                p1 = r0_v[b, r, pl.ds(16, 16)] * r1_v[b, r, pl.ds(16, 16)]
                acc_v[b, r // 8, pl.ds((r % 8) * 16, 16)] = p0 + p1

            store_start(b, c)
            load_start(b, c + 2)
            nb = 1 - b
            load_wait(nb)
            gather_start(nb)

        load_start(0, 0)
        load_start(1, 1)
        load_wait(0)
        gather_start(0)
        half(0, 0, True)
        half(1, 1, True)

        @pl.loop(1, NCH_D // 2)
        def _(gi):
            half(2 * gi, 0, False)
            half(2 * gi + 1, 1, False)

        gather_wait(0)
        load_wait(1)
        store_wait(0)
        store_wait(1)

    return scd(z, eid)


def kernel(x, edge_index, pos_edge_index, neg_edge_index,
           W1, a1_src, a1_dst, b1, W2, a2_src, a2_dst, b2):
    loops = jnp.arange(N, dtype=edge_index.dtype)
    ei = jnp.concatenate(
        [edge_index, jnp.stack([loops, loops], axis=0)], axis=1)
    pad = E_PAD - E
    ei_pad = jnp.pad(ei, ((0, 0), (0, pad)))

    dec = jnp.concatenate([pos_edge_index, neg_edge_index], axis=1)
    dpad = P_PAD - P
    dec_pad = jnp.pad(dec, ((0, 0), (0, dpad)))

    h1, as1, ad1, c1 = _tc_embed(x, W1, a1_src, a1_dst)
    outp1, dp1 = _sc_layer(h1, as1, ad1, c1, ei_pad)

    h2, as2, ad2, c2 = _tc_mid(outp1, dp1.reshape(2, NPAD), b1,
                               W2, a2_src, a2_dst)
    outp2, dp2 = _sc_layer(h2, as2, ad2, c2, ei_pad)

    z = _tc_z(outp2, dp2.reshape(2, NPAD), b2)
    acc = _sc_decode(z, dec_pad)
    logits_pad = _tc_fold(acc)
    return logits_pad.reshape(P_PAD)[:P]


# speedup vs baseline: 1.0865x; 1.0809x over previous
import dataclasses
import functools

import jax
import jax.numpy as jnp
from jax import lax
from jax.experimental import pallas as pl
from jax.experimental.pallas import tpu as pltpu
from jax.experimental.pallas import tpu_sc as plsc

N = 10000
NPAD = 10240
E = 330000
NTILES = 32
NCH = 84
T_EDGE = NCH * 128
E_PAD = NTILES * T_EDGE
P = 200000
NCH_D = 50
T_DEC = NCH_D * 128
P_PAD = NTILES * T_DEC

_MESH = plsc.VectorSubcoreMesh(core_axis_name="c", subcore_axis_name="s")

_SC_PARAMS = pltpu.CompilerParams()
if "needs_layout_passes" in pltpu.CompilerParams.__dataclass_fields__:
    _SC_PARAMS = dataclasses.replace(_SC_PARAMS, needs_layout_passes=False)
if "use_tc_tiling_on_sc" in pltpu.CompilerParams.__dataclass_fields__:
    _SC_PARAMS = dataclasses.replace(_SC_PARAMS, use_tc_tiling_on_sc=False)


def _tc_embed_body(x_ref, w_ref, av_ref, bv_ref, h_ref, as_ref, ad_ref, c_ref):
    h = jnp.dot(x_ref[...], w_ref[...], preferred_element_type=jnp.float32)
    h_ref[...] = h
    asrc = jnp.sum(h * av_ref[...][None, :], axis=1)
    adst = jnp.sum(h * bv_ref[...][None, :], axis=1)
    as_ref[...] = asrc
    ad_ref[...] = adst
    c_ref[...] = jnp.broadcast_to(jnp.max(asrc) + jnp.max(adst), (16,))


def _tc_embed(x, w, avec, bvec):
    m = x.shape[0]
    d = w.shape[1]
    return pl.pallas_call(
        _tc_embed_body,
        out_shape=[
            jax.ShapeDtypeStruct((m, d), jnp.float32),
            jax.ShapeDtypeStruct((m,), jnp.float32),
            jax.ShapeDtypeStruct((m,), jnp.float32),
            jax.ShapeDtypeStruct((16,), jnp.float32),
        ],
    )(x, w, avec, bvec)


def _tc_mid_body(p_ref, dp_ref, b_ref, w_ref, av_ref, bv_ref,
                 h_ref, as_ref, ad_ref, c_ref):
    den = dp_ref[0] + dp_ref[1] + 1e-16
    t = (p_ref[0] + p_ref[1]) / den[:, None] + b_ref[...][None, :]
    t = jnp.maximum(t, 0.0)
    h = jnp.dot(t, w_ref[...], preferred_element_type=jnp.float32)
    h_ref[...] = h
    asrc = jnp.sum(h * av_ref[...][None, :], axis=1)
    adst = jnp.sum(h * bv_ref[...][None, :], axis=1)
    as_ref[...] = asrc
    ad_ref[...] = adst
    c_ref[...] = jnp.broadcast_to(jnp.max(asrc) + jnp.max(adst), (16,))


def _tc_mid(parts, dp, b, w, avec, bvec):
    m = parts.shape[1]
    d = w.shape[1]
    return pl.pallas_call(
        _tc_mid_body,
        out_shape=[
            jax.ShapeDtypeStruct((m, d), jnp.float32),
            jax.ShapeDtypeStruct((m,), jnp.float32),
            jax.ShapeDtypeStruct((m,), jnp.float32),
            jax.ShapeDtypeStruct((16,), jnp.float32),
        ],
    )(parts, dp, b, w, avec, bvec)


def _tc_z_body(p_ref, dp_ref, b_ref, z_ref):
    den = dp_ref[0] + dp_ref[1] + 1e-16
    z_ref[...] = (p_ref[0] + p_ref[1]) / den[:, None] + b_ref[...][None, :]


def _tc_z(parts, dp, b):
    return pl.pallas_call(
        _tc_z_body,
        out_shape=jax.ShapeDtypeStruct(parts.shape[1:], jnp.float32),
    )(parts, dp, b)


def _tc_fold_body(a_ref, o_ref):
    lanes = lax.broadcasted_iota(jnp.int32, (128, 8), 0)
    cols = lax.broadcasted_iota(jnp.int32, (128, 8), 1)
    s = jnp.where(lanes // 16 == cols, 1.0, 0.0)
    o_ref[...] = jnp.dot(a_ref[...], s, preferred_element_type=jnp.float32)


def _tc_fold(acc):
    return pl.pallas_call(
        _tc_fold_body,
        out_shape=jax.ShapeDtypeStruct((acc.shape[0], 8), jnp.float32),
    )(acc)


def _sc_layer(h, asrc, adst, cvec, src, dst):
    m = asrc.shape[0]
    d = h.shape[1]
    ng = d // 16

    @functools.partial(
        pl.kernel,
        out_type=[
            jax.ShapeDtypeStruct((2, NPAD, d), jnp.float32),
            jax.ShapeDtypeStruct((2 * NPAD,), jnp.float32),
        ],
        mesh=_MESH,
        compiler_params=_SC_PARAMS,
        scratch_types=[
            pltpu.VMEM((NPAD,), jnp.float32),
            pltpu.VMEM((NPAD,), jnp.float32),
            pltpu.VMEM((16,), jnp.float32),
            pltpu.VMEM((4, 128), jnp.int32),
            pltpu.VMEM((4, 128), jnp.int32),
            pltpu.VMEM((512,), jnp.float32),
            pltpu.VMEM((2, 128, d), jnp.float32),
            pltpu.VMEM((640,), jnp.float32),
            pltpu.VMEM((NPAD,), jnp.float32),
            pltpu.VMEM((16, 640), jnp.float32),
            pltpu.VMEM_SHARED((NPAD, d), jnp.float32),
            pltpu.VMEM_SHARED((16, NPAD), jnp.float32),
            pltpu.SemaphoreType.DMA,
            pltpu.SemaphoreType.DMA,
            pltpu.SemaphoreType.DMA,
            pltpu.SemaphoreType.DMA,
            pltpu.SemaphoreType.DMA,
            pltpu.SemaphoreType.DMA,
            pltpu.SemaphoreType.DMA,
            pltpu.SemaphoreType.DMA,
        ],
    )
    def scl(h_h, as_h, ad_h, c_h, src_h, dst_h, outp_h, dp_h,
            asrc_v, adst_v, c_v, src_v, dst_v, w_v, rows_v, zbuf,
            denom_v, stage_v, out_sh, dred_sh,
            ld0, ld1, ld2, ld3, g0, g1, sc0, sc1):
        cid = lax.axis_index("c")
        sid = lax.axis_index("s")
        wid = sid * 2 + cid
        base = wid * T_EDGE
        ld_sems = [ld0, ld1, ld2, ld3]
        g_sems = [g0, g1]
        sc_sems = [sc0, sc1]

        @pl.loop(0, 128)
        def _(r):
            for g_ in range(ng):
                rows_v[0, r, pl.ds(g_ * 16, 16)] = jnp.zeros((16,), jnp.float32)

        @pl.loop(0, NPAD // 16)
        def _(i):
            denom_v[pl.ds(i * 16, 16)] = jnp.zeros((16,), jnp.float32)

        @pl.loop(0, 5)
        def _(i):
            pltpu.sync_copy(rows_v.at[0],
                            out_sh.at[pl.ds((sid * 5 + i) * 128, 128)])

        pltpu.sync_copy(as_h.at[pl.ds(0, m)], asrc_v.at[pl.ds(0, m)])
        pltpu.sync_copy(ad_h.at[pl.ds(0, m)], adst_v.at[pl.ds(0, m)])
        pltpu.sync_copy(c_h, c_v)
        plsc.subcore_barrier()

        def load_start(b4, c):
            off = pl.multiple_of(
                jnp.minimum(base + c * 128, E_PAD - 128), 128)
            pltpu.make_async_copy(
                src_h.at[pl.ds(off, 128)], src_v.at[b4], ld_sems[b4]).start()
            pltpu.make_async_copy(
                dst_h.at[pl.ds(off, 128)], dst_v.at[b4], ld_sems[b4]).start()

        def load_wait(b4):
            pltpu.make_async_copy(
                src_h.at[pl.ds(0, 128)], src_v.at[b4], ld_sems[b4]).wait()
            pltpu.make_async_copy(
                dst_h.at[pl.ds(0, 128)], dst_v.at[b4], ld_sems[b4]).wait()

        def gather_start(b4, b2):
            pltpu.make_async_copy(
                h_h.at[src_v.at[b4]], rows_v.at[b2], g_sems[b2]).start()

        def gather_wait(b4, b2):
            pltpu.make_async_copy(
                h_h.at[src_v.at[b4]], rows_v.at[b2], g_sems[b2]).wait()

        def scatter_start(b4, b2):
            pltpu.async_copy(rows_v.at[b2], out_sh.at[dst_v.at[b4]],
                             sc_sems[b2], add=True)

        def scatter_wait(b4, b2):
            pltpu.make_async_copy(
                rows_v.at[b2], out_sh.at[dst_v.at[b4]], sc_sems[b2]).wait()

        def compute_w(b4, c):
            cb = base + c * 128
            for k in range(8):
                sv = src_v[b4, pl.ds(k * 16, 16)]
                dv = dst_v[b4, pl.ds(k * 16, 16)]
                av = plsc.load_gather(asrc_v, [sv])
                bv = plsc.load_gather(adst_v, [dv])
                e = av + bv
                e = jnp.where(e >= 0.0, e, 0.2 * e)
                wv = jnp.exp(e - c_v[...])
                gid = lax.iota(jnp.int32, 16) + (cb + k * 16)
                wv = jnp.where(gid < E, wv, 0.0)
                w_v[pl.ds(b4 * 128 + k * 16, 16)] = wv
                plsc.addupdate_scatter(denom_v, [dv], wv)

        def scale(b4, b2):
            @plsc.parallel_loop(0, 128, unroll=8)
            def _(r):
                widx = jnp.full((16,), b4 * 128, jnp.int32) + r
                wvec = plsc.load_gather(w_v, [widx])
                for g_ in range(ng):
                    rows_v[b2, r, pl.ds(g_ * 16, 16)] = (
                        rows_v[b2, r, pl.ds(g_ * 16, 16)] * wvec)

        def half(c, b4, b2, first):
            gather_wait(b4, b2)
            compute_w(b4, c)
            scale(b4, b2)
            scatter_start(b4, b2)
            load_start((b4 + 2) % 4, c + 2)
            nb4 = (b4 + 1) % 4
            nb2 = 1 - b2
            load_wait(nb4)
            if not first:
                scatter_wait((b4 + 3) % 4, nb2)
            gather_start(nb4, nb2)

        load_start(0, 0)
        load_start(1, 1)
        load_wait(0)
        gather_start(0, 0)
        half(0, 0, 0, True)
        half(1, 1, 1, False)
        half(2, 2, 0, False)
        half(3, 3, 1, False)

        @pl.loop(1, (NCH - 4) // 4 + 1)
        def _(gi):
            c0 = 4 * gi
            half(c0, 0, 0, False)
            half(c0 + 1, 1, 1, False)
            half(c0 + 2, 2, 0, False)
            half(c0 + 3, 3, 1, False)

        gather_wait(0, 0)
        load_wait(1)
        scatter_wait(3, 1)

        pltpu.sync_copy(denom_v, dred_sh.at[sid])
        plsc.subcore_barrier()

        @pl.loop(0, 16)
        def _(t):
            pltpu.sync_copy(dred_sh.at[t, pl.ds(sid * 640, 640)],
                            stage_v.at[t])

        @pl.loop(0, 40)
        def _(j):
            tot = stage_v[0, pl.ds(j * 16, 16)]
            for t in range(1, 16):
                tot = tot + stage_v[t, pl.ds(j * 16, 16)]
            zbuf[pl.ds(j * 16, 16)] = tot

        @pl.loop(0, 5)
        def _(i):
            s = (sid * 5 + i) * 128
            pltpu.sync_copy(out_sh.at[pl.ds(s, 128)],
                            outp_h.at[cid, pl.ds(s, 128)])

        pltpu.sync_copy(zbuf, dp_h.at[pl.ds(cid * NPAD + sid * 640, 640)])

    return scl(h, asrc, adst, cvec, src, dst)


def _sc_decode(z, ei0, ei1):
    @functools.partial(
        pl.kernel,
        out_type=jax.ShapeDtypeStruct((P_PAD // 8, 128), jnp.float32),
        mesh=_MESH,
        compiler_params=_SC_PARAMS,
        scratch_types=[
            pltpu.VMEM((2, 128), jnp.int32),
            pltpu.VMEM((2, 128), jnp.int32),
            pltpu.VMEM((2, 128, 32), jnp.float32),
            pltpu.VMEM((2, 128, 32), jnp.float32),
            pltpu.VMEM((2, 16, 128), jnp.float32),
            pltpu.SemaphoreType.DMA,
            pltpu.SemaphoreType.DMA,
            pltpu.SemaphoreType.DMA,
            pltpu.SemaphoreType.DMA,
            pltpu.SemaphoreType.DMA,
            pltpu.SemaphoreType.DMA,
        ],
    )
    def scd(z_h, e0_h, e1_h, acc_h,
            i0_v, i1_v, r0_v, r1_v, acc_v, ld0, ld1, g0, g1, st0, st1):
        cid = lax.axis_index("c")
        sid = lax.axis_index("s")
        wid = sid * 2 + cid
        base = wid * T_DEC
        ld_sems = [ld0, ld1]
        g_sems = [g0, g1]
        st_sems = [st0, st1]

        def load_start(b, c):
            off = pl.multiple_of(
                jnp.minimum(base + c * 128, P_PAD - 128), 128)
            pltpu.make_async_copy(
                e0_h.at[pl.ds(off, 128)], i0_v.at[b], ld_sems[b]).start()
            pltpu.make_async_copy(
                e1_h.at[pl.ds(off, 128)], i1_v.at[b], ld_sems[b]).start()

        def load_wait(b):
            pltpu.make_async_copy(
                e0_h.at[pl.ds(0, 128)], i0_v.at[b], ld_sems[b]).wait()
            pltpu.make_async_copy(
                e1_h.at[pl.ds(0, 128)], i1_v.at[b], ld_sems[b]).wait()

        def gather_start(b):
            pltpu.make_async_copy(
                z_h.at[i0_v.at[b]], r0_v.at[b], g_sems[b]).start()
            pltpu.make_async_copy(
                z_h.at[i1_v.at[b]], r1_v.at[b], g_sems[b]).start()

        def gather_wait(b):
            pltpu.make_async_copy(
                z_h.at[i0_v.at[b]], r0_v.at[b], g_sems[b]).wait()
            pltpu.make_async_copy(
                z_h.at[i1_v.at[b]], r1_v.at[b], g_sems[b]).wait()

        def store_start(b, c):
            off = pl.multiple_of(base // 8 + c * 16, 16)
            pltpu.make_async_copy(
                acc_v.at[b], acc_h.at[pl.ds(off, 16)], st_sems[b]).start()

        def store_wait(b):
            pltpu.make_async_copy(
                acc_v.at[b], acc_h.at[pl.ds(0, 16)], st_sems[b]).wait()

        def half(c, b, first):
            gather_wait(b)
            if not first:
                store_wait(b)

            @plsc.parallel_loop(0, 128, unroll=8)
            def _(r):
                p0 = r0_v[b, r, pl.ds(0, 16)] * r1_v[b, r, pl.ds(0, 16)]
                p1 = r0_v[b, r, pl.ds(16, 16)] * r1_v[b, r, pl.ds(16, 16)]
                acc_v[b, r // 8, pl.ds((r % 8) * 16, 16)] = p0 + p1

            store_start(b, c)
            load_start(b, c + 2)
            nb = 1 - b
            load_wait(nb)
            gather_start(nb)

        load_start(0, 0)
        load_start(1, 1)
        load_wait(0)
        gather_start(0)
        half(0, 0, True)
        half(1, 1, True)

        @pl.loop(1, NCH_D // 2)
        def _(gi):
            half(2 * gi, 0, False)
            half(2 * gi + 1, 1, False)

        gather_wait(0)
        load_wait(1)
        store_wait(0)
        store_wait(1)

    return scd(z, ei0, ei1)


def kernel(x, edge_index, pos_edge_index, neg_edge_index,
           W1, a1_src, a1_dst, b1, W2, a2_src, a2_dst, b2):
    loops = jnp.arange(N, dtype=edge_index.dtype)
    ei = jnp.concatenate(
        [edge_index, jnp.stack([loops, loops], axis=0)], axis=1)
    pad = E_PAD - E
    src = jnp.pad(ei[0], (0, pad))
    dst = jnp.pad(ei[1], (0, pad))

    dec = jnp.concatenate([pos_edge_index, neg_edge_index], axis=1)
    dpad = P_PAD - P
    d0 = jnp.pad(dec[0], (0, dpad))
    d1 = jnp.pad(dec[1], (0, dpad))

    h1, as1, ad1, c1 = _tc_embed(x, W1, a1_src, a1_dst)
    outp1, dp1 = _sc_layer(h1, as1, ad1, c1, src, dst)

    h2, as2, ad2, c2 = _tc_mid(outp1, dp1.reshape(2, NPAD), b1,
                               W2, a2_src, a2_dst)
    outp2, dp2 = _sc_layer(h2, as2, ad2, c2, src, dst)

    z = _tc_z(outp2, dp2.reshape(2, NPAD), b2)
    acc = _sc_decode(z, d0, d1)
    logits_pad = _tc_fold(acc)
    return logits_pad.reshape(P_PAD)[:P]


# speedup vs baseline: 1.5469x; 1.4238x over previous
import dataclasses
import functools

import jax
import jax.numpy as jnp
from jax import lax
from jax.experimental import pallas as pl
from jax.experimental.pallas import tpu as pltpu
from jax.experimental.pallas import tpu_sc as plsc

N = 10000
NPAD = 10240
E = 330000
NTILES = 32
NCH = 84
T_EDGE = NCH * 128
E_PAD = NTILES * T_EDGE
P = 200000
NCH_D = 50
T_DEC = NCH_D * 128
P_PAD = NTILES * T_DEC

_MESH = plsc.VectorSubcoreMesh(core_axis_name="c", subcore_axis_name="s")

_SC_PARAMS = pltpu.CompilerParams()
if "needs_layout_passes" in pltpu.CompilerParams.__dataclass_fields__:
    _SC_PARAMS = dataclasses.replace(_SC_PARAMS, needs_layout_passes=False)
if "use_tc_tiling_on_sc" in pltpu.CompilerParams.__dataclass_fields__:
    _SC_PARAMS = dataclasses.replace(_SC_PARAMS, use_tc_tiling_on_sc=False)


def _tc_embed_body(x_ref, w_ref, av_ref, bv_ref, h_ref, as_ref, ad_ref, c_ref):
    h = jnp.dot(x_ref[...], w_ref[...], preferred_element_type=jnp.float32)
    h_ref[...] = h
    asrc = jnp.sum(h * av_ref[...][None, :], axis=1)
    adst = jnp.sum(h * bv_ref[...][None, :], axis=1)
    as_ref[...] = asrc
    ad_ref[...] = adst
    c_ref[...] = jnp.broadcast_to(jnp.max(asrc) + jnp.max(adst), (16,))


def _tc_embed(x, w, avec, bvec):
    m = x.shape[0]
    d = w.shape[1]
    return pl.pallas_call(
        _tc_embed_body,
        out_shape=[
            jax.ShapeDtypeStruct((m, d), jnp.float32),
            jax.ShapeDtypeStruct((m,), jnp.float32),
            jax.ShapeDtypeStruct((m,), jnp.float32),
            jax.ShapeDtypeStruct((16,), jnp.float32),
        ],
    )(x, w, avec, bvec)


def _tc_mid_body(p_ref, dp_ref, b_ref, w_ref, av_ref, bv_ref,
                 h_ref, as_ref, ad_ref, c_ref):
    den = dp_ref[0] + dp_ref[1] + 1e-16
    t = (p_ref[0] + p_ref[1]) / den[:, None] + b_ref[...][None, :]
    t = jnp.maximum(t, 0.0)
    h = jnp.dot(t, w_ref[...], preferred_element_type=jnp.float32)
    h_ref[...] = h
    asrc = jnp.sum(h * av_ref[...][None, :], axis=1)
    adst = jnp.sum(h * bv_ref[...][None, :], axis=1)
    as_ref[...] = asrc
    ad_ref[...] = adst
    c_ref[...] = jnp.broadcast_to(jnp.max(asrc) + jnp.max(adst), (16,))


def _tc_mid(parts, dp, b, w, avec, bvec):
    m = parts.shape[1]
    d = w.shape[1]
    return pl.pallas_call(
        _tc_mid_body,
        out_shape=[
            jax.ShapeDtypeStruct((m, d), jnp.float32),
            jax.ShapeDtypeStruct((m,), jnp.float32),
            jax.ShapeDtypeStruct((m,), jnp.float32),
            jax.ShapeDtypeStruct((16,), jnp.float32),
        ],
    )(parts, dp, b, w, avec, bvec)


def _tc_z_body(p_ref, dp_ref, b_ref, z_ref):
    den = dp_ref[0] + dp_ref[1] + 1e-16
    z_ref[...] = (p_ref[0] + p_ref[1]) / den[:, None] + b_ref[...][None, :]


def _tc_z(parts, dp, b):
    return pl.pallas_call(
        _tc_z_body,
        out_shape=jax.ShapeDtypeStruct(parts.shape[1:], jnp.float32),
    )(parts, dp, b)


def _tc_fold_body(a_ref, o_ref):
    lanes = lax.broadcasted_iota(jnp.int32, (128, 8), 0)
    cols = lax.broadcasted_iota(jnp.int32, (128, 8), 1)
    s = jnp.where(lanes // 16 == cols, 1.0, 0.0)
    o_ref[...] = jnp.dot(a_ref[...], s, preferred_element_type=jnp.float32)


def _tc_fold(acc):
    return pl.pallas_call(
        _tc_fold_body,
        out_shape=jax.ShapeDtypeStruct((acc.shape[0], 8), jnp.float32),
    )(acc)


def _sc_layer(h, asrc, adst, cvec, src, dst, stage_h):
    m = asrc.shape[0]
    d = h.shape[1]
    ng = d // 16

    @functools.partial(
        pl.kernel,
        out_type=[
            jax.ShapeDtypeStruct((2, NPAD, d), jnp.float32),
            jax.ShapeDtypeStruct((2 * NPAD,), jnp.float32),
        ],
        mesh=_MESH,
        compiler_params=_SC_PARAMS,
        scratch_types=[
            pltpu.VMEM((NPAD,), jnp.float32),
            pltpu.VMEM((NPAD,), jnp.float32),
            pltpu.VMEM((16,), jnp.float32),
            pltpu.VMEM((4, 128), jnp.int32),
            pltpu.VMEM((4, 128), jnp.int32),
            pltpu.VMEM((512,), jnp.float32),
            pltpu.VMEM((2, 128, d), jnp.float32),
            pltpu.VMEM((640,), jnp.float32),
            pltpu.VMEM((NPAD,), jnp.float32),
            pltpu.VMEM((16, 640), jnp.float32),
            pltpu.VMEM_SHARED((NPAD, d), jnp.float32),
            pltpu.VMEM_SHARED((16, NPAD), jnp.float32),
            pltpu.VMEM_SHARED((NPAD, d) if stage_h else (16,),
                              jnp.float32),
            pltpu.SemaphoreType.DMA,
            pltpu.SemaphoreType.DMA,
            pltpu.SemaphoreType.DMA,
            pltpu.SemaphoreType.DMA,
            pltpu.SemaphoreType.DMA,
            pltpu.SemaphoreType.DMA,
            pltpu.SemaphoreType.DMA,
            pltpu.SemaphoreType.DMA,
        ],
    )
    def scl(h_h, as_h, ad_h, c_h, src_h, dst_h, outp_h, dp_h,
            asrc_v, adst_v, c_v, src_v, dst_v, w_v, rows_v, zbuf,
            denom_v, stage_v, out_sh, dred_sh, h_sh,
            ld0, ld1, ld2, ld3, g0, g1, sc0, sc1):
        cid = lax.axis_index("c")
        sid = lax.axis_index("s")
        wid = sid * 2 + cid
        base = wid * T_EDGE
        ld_sems = [ld0, ld1, ld2, ld3]
        g_sems = [g0, g1]
        sc_sems = [sc0, sc1]

        @pl.loop(0, 128)
        def _(r):
            for g_ in range(ng):
                rows_v[0, r, pl.ds(g_ * 16, 16)] = jnp.zeros((16,), jnp.float32)

        @pl.loop(0, NPAD // 16)
        def _(i):
            denom_v[pl.ds(i * 16, 16)] = jnp.zeros((16,), jnp.float32)

        @pl.loop(0, 5)
        def _(i):
            pltpu.sync_copy(rows_v.at[0],
                            out_sh.at[pl.ds((sid * 5 + i) * 128, 128)])

        pltpu.sync_copy(as_h.at[pl.ds(0, m)], asrc_v.at[pl.ds(0, m)])
        pltpu.sync_copy(ad_h.at[pl.ds(0, m)], adst_v.at[pl.ds(0, m)])
        pltpu.sync_copy(c_h, c_v)
        if stage_h:
            pltpu.sync_copy(h_h.at[pl.ds(sid * 640, 640)],
                            h_sh.at[pl.ds(sid * 640, 640)])
        plsc.subcore_barrier()

        def load_start(b4, c):
            off = pl.multiple_of(
                jnp.minimum(base + c * 128, E_PAD - 128), 128)
            pltpu.make_async_copy(
                src_h.at[pl.ds(off, 128)], src_v.at[b4], ld_sems[b4]).start()
            pltpu.make_async_copy(
                dst_h.at[pl.ds(off, 128)], dst_v.at[b4], ld_sems[b4]).start()

        def load_wait(b4):
            pltpu.make_async_copy(
                src_h.at[pl.ds(0, 128)], src_v.at[b4], ld_sems[b4]).wait()
            pltpu.make_async_copy(
                dst_h.at[pl.ds(0, 128)], dst_v.at[b4], ld_sems[b4]).wait()

        h_src = h_sh if stage_h else h_h

        def gather_start(b4, b2):
            pltpu.make_async_copy(
                h_src.at[src_v.at[b4]], rows_v.at[b2], g_sems[b2]).start()

        def gather_wait(b4, b2):
            pltpu.make_async_copy(
                h_src.at[src_v.at[b4]], rows_v.at[b2], g_sems[b2]).wait()

        def scatter_start(b4, b2):
            pltpu.async_copy(rows_v.at[b2], out_sh.at[dst_v.at[b4]],
                             sc_sems[b2], add=True)

        def scatter_wait(b4, b2):
            pltpu.make_async_copy(
                rows_v.at[b2], out_sh.at[dst_v.at[b4]], sc_sems[b2]).wait()

        def compute_w(b4, c):
            cb = base + c * 128
            for k in range(8):
                sv = src_v[b4, pl.ds(k * 16, 16)]
                dv = dst_v[b4, pl.ds(k * 16, 16)]
                av = plsc.load_gather(asrc_v, [sv])
                bv = plsc.load_gather(adst_v, [dv])
                e = av + bv
                e = jnp.where(e >= 0.0, e, 0.2 * e)
                wv = jnp.exp(e - c_v[...])
                gid = lax.iota(jnp.int32, 16) + (cb + k * 16)
                wv = jnp.where(gid < E, wv, 0.0)
                w_v[pl.ds(b4 * 128 + k * 16, 16)] = wv
                plsc.addupdate_scatter(denom_v, [dv], wv)

        def scale(b4, b2):
            @plsc.parallel_loop(0, 128, unroll=8)
            def _(r):
                widx = jnp.full((16,), b4 * 128, jnp.int32) + r
                wvec = plsc.load_gather(w_v, [widx])
                for g_ in range(ng):
                    rows_v[b2, r, pl.ds(g_ * 16, 16)] = (
                        rows_v[b2, r, pl.ds(g_ * 16, 16)] * wvec)

        def half(c, b4, b2, first):
            gather_wait(b4, b2)
            compute_w(b4, c)
            scale(b4, b2)
            scatter_start(b4, b2)
            load_start((b4 + 2) % 4, c + 2)
            nb4 = (b4 + 1) % 4
            nb2 = 1 - b2
            load_wait(nb4)
            if not first:
                scatter_wait((b4 + 3) % 4, nb2)
            gather_start(nb4, nb2)

        load_start(0, 0)
        load_start(1, 1)
        load_wait(0)
        gather_start(0, 0)
        half(0, 0, 0, True)
        half(1, 1, 1, False)
        half(2, 2, 0, False)
        half(3, 3, 1, False)

        @pl.loop(1, (NCH - 4) // 4 + 1)
        def _(gi):
            c0 = 4 * gi
            half(c0, 0, 0, False)
            half(c0 + 1, 1, 1, False)
            half(c0 + 2, 2, 0, False)
            half(c0 + 3, 3, 1, False)

        gather_wait(0, 0)
        load_wait(1)
        scatter_wait(3, 1)

        pltpu.sync_copy(denom_v, dred_sh.at[sid])
        plsc.subcore_barrier()

        @pl.loop(0, 16)
        def _(t):
            pltpu.sync_copy(dred_sh.at[t, pl.ds(sid * 640, 640)],
                            stage_v.at[t])

        @pl.loop(0, 40)
        def _(j):
            tot = stage_v[0, pl.ds(j * 16, 16)]
            for t in range(1, 16):
                tot = tot + stage_v[t, pl.ds(j * 16, 16)]
            zbuf[pl.ds(j * 16, 16)] = tot

        @pl.loop(0, 5)
        def _(i):
            s = (sid * 5 + i) * 128
            pltpu.sync_copy(out_sh.at[pl.ds(s, 128)],
                            outp_h.at[cid, pl.ds(s, 128)])

        pltpu.sync_copy(zbuf, dp_h.at[pl.ds(cid * NPAD + sid * 640, 640)])

    return scl(h, asrc, adst, cvec, src, dst)


def _sc_decode(z, ei0, ei1):
    @functools.partial(
        pl.kernel,
        out_type=jax.ShapeDtypeStruct((P_PAD // 8, 128), jnp.float32),
        mesh=_MESH,
        compiler_params=_SC_PARAMS,
        scratch_types=[
            pltpu.VMEM((2, 128), jnp.int32),
            pltpu.VMEM((2, 128), jnp.int32),
            pltpu.VMEM((2, 128, 32), jnp.float32),
            pltpu.VMEM((2, 128, 32), jnp.float32),
            pltpu.VMEM((2, 16, 128), jnp.float32),
            pltpu.VMEM_SHARED((NPAD, 32), jnp.float32),
            pltpu.SemaphoreType.DMA,
            pltpu.SemaphoreType.DMA,
            pltpu.SemaphoreType.DMA,
            pltpu.SemaphoreType.DMA,
            pltpu.SemaphoreType.DMA,
            pltpu.SemaphoreType.DMA,
        ],
    )
    def scd(z_h, e0_h, e1_h, acc_h,
            i0_v, i1_v, r0_v, r1_v, acc_v, z_sh,
            ld0, ld1, g0, g1, st0, st1):
        cid = lax.axis_index("c")
        sid = lax.axis_index("s")
        wid = sid * 2 + cid
        base = wid * T_DEC
        ld_sems = [ld0, ld1]
        g_sems = [g0, g1]
        st_sems = [st0, st1]

        pltpu.sync_copy(z_h.at[pl.ds(sid * 640, 640)],
                        z_sh.at[pl.ds(sid * 640, 640)])
        plsc.subcore_barrier()

        def load_start(b, c):
            off = pl.multiple_of(
                jnp.minimum(base + c * 128, P_PAD - 128), 128)
            pltpu.make_async_copy(
                e0_h.at[pl.ds(off, 128)], i0_v.at[b], ld_sems[b]).start()
            pltpu.make_async_copy(
                e1_h.at[pl.ds(off, 128)], i1_v.at[b], ld_sems[b]).start()

        def load_wait(b):
            pltpu.make_async_copy(
                e0_h.at[pl.ds(0, 128)], i0_v.at[b], ld_sems[b]).wait()
            pltpu.make_async_copy(
                e1_h.at[pl.ds(0, 128)], i1_v.at[b], ld_sems[b]).wait()

        def gather_start(b):
            pltpu.make_async_copy(
                z_sh.at[i0_v.at[b]], r0_v.at[b], g_sems[b]).start()
            pltpu.make_async_copy(
                z_sh.at[i1_v.at[b]], r1_v.at[b], g_sems[b]).start()

        def gather_wait(b):
            pltpu.make_async_copy(
                z_sh.at[i0_v.at[b]], r0_v.at[b], g_sems[b]).wait()
            pltpu.make_async_copy(
                z_sh.at[i1_v.at[b]], r1_v.at[b], g_sems[b]).wait()

        def store_start(b, c):
            off = pl.multiple_of(base // 8 + c * 16, 16)
            pltpu.make_async_copy(
                acc_v.at[b], acc_h.at[pl.ds(off, 16)], st_sems[b]).start()

        def store_wait(b):
            pltpu.make_async_copy(
                acc_v.at[b], acc_h.at[pl.ds(0, 16)], st_sems[b]).wait()

        def half(c, b, first):
            gather_wait(b)
            if not first:
                store_wait(b)

            @plsc.parallel_loop(0, 128, unroll=8)
            def _(r):
                p0 = r0_v[b, r, pl.ds(0, 16)] * r1_v[b, r, pl.ds(0, 16)]
                p1 = r0_v[b, r, pl.ds(16, 16)] * r1_v[b, r, pl.ds(16, 16)]
                acc_v[b, r // 8, pl.ds((r % 8) * 16, 16)] = p0 + p1

            store_start(b, c)
            load_start(b, c + 2)
            nb = 1 - b
            load_wait(nb)
            gather_start(nb)

        load_start(0, 0)
        load_start(1, 1)
        load_wait(0)
        gather_start(0)
        half(0, 0, True)
        half(1, 1, True)

        @pl.loop(1, NCH_D // 2)
        def _(gi):
            half(2 * gi, 0, False)
            half(2 * gi + 1, 1, False)

        gather_wait(0)
        load_wait(1)
        store_wait(0)
        store_wait(1)

    return scd(z, ei0, ei1)


def kernel(x, edge_index, pos_edge_index, neg_edge_index,
           W1, a1_src, a1_dst, b1, W2, a2_src, a2_dst, b2):
    loops = jnp.arange(N, dtype=edge_index.dtype)
    ei = jnp.concatenate(
        [edge_index, jnp.stack([loops, loops], axis=0)], axis=1)
    pad = E_PAD - E
    src = jnp.pad(ei[0], (0, pad))
    dst = jnp.pad(ei[1], (0, pad))

    dec = jnp.concatenate([pos_edge_index, neg_edge_index], axis=1)
    dpad = P_PAD - P
    d0 = jnp.pad(dec[0], (0, dpad))
    d1 = jnp.pad(dec[1], (0, dpad))

    h1, as1, ad1, c1 = _tc_embed(x, W1, a1_src, a1_dst)
    outp1, dp1 = _sc_layer(h1, as1, ad1, c1, src, dst, stage_h=False)

    h2, as2, ad2, c2 = _tc_mid(outp1, dp1.reshape(2, NPAD), b1,
                               W2, a2_src, a2_dst)
    outp2, dp2 = _sc_layer(h2, as2, ad2, c2, src, dst, stage_h=True)

    z = _tc_z(outp2, dp2.reshape(2, NPAD), b2)
    acc = _sc_decode(z, d0, d1)
    logits_pad = _tc_fold(acc)
    return logits_pad.reshape(P_PAD)[:P]


# speedup vs baseline: 2.6715x; 1.7270x over previous
import dataclasses
import functools

import jax
import jax.numpy as jnp
from jax import lax
from jax.experimental import pallas as pl
from jax.experimental.pallas import tpu as pltpu
from jax.experimental.pallas import tpu_sc as plsc

N = 10000
NPAD = 10240
E = 330000
NTILES = 32
NCH = 84
T_EDGE = NCH * 128
E_PAD = NTILES * T_EDGE
P = 200000
NCH_D = 50
T_DEC = NCH_D * 128
P_PAD = NTILES * T_DEC

_MESH = plsc.VectorSubcoreMesh(core_axis_name="c", subcore_axis_name="s")

_SC_PARAMS = pltpu.CompilerParams()
if "needs_layout_passes" in pltpu.CompilerParams.__dataclass_fields__:
    _SC_PARAMS = dataclasses.replace(_SC_PARAMS, needs_layout_passes=False)
if "use_tc_tiling_on_sc" in pltpu.CompilerParams.__dataclass_fields__:
    _SC_PARAMS = dataclasses.replace(_SC_PARAMS, use_tc_tiling_on_sc=False)


def _tc_embed_body(x_ref, w_ref, av_ref, bv_ref, h_ref, as_ref, ad_ref, c_ref):
    h = jnp.dot(x_ref[...], w_ref[...], preferred_element_type=jnp.float32)
    h_ref[...] = h
    asrc = jnp.sum(h * av_ref[...][None, :], axis=1)
    adst = jnp.sum(h * bv_ref[...][None, :], axis=1)
    as_ref[...] = asrc
    ad_ref[...] = adst
    c_ref[...] = jnp.broadcast_to(jnp.max(asrc) + jnp.max(adst), (16,))


def _tc_embed(x, w, avec, bvec):
    m = x.shape[0]
    d = w.shape[1]
    return pl.pallas_call(
        _tc_embed_body,
        out_shape=[
            jax.ShapeDtypeStruct((m, d), jnp.float32),
            jax.ShapeDtypeStruct((m,), jnp.float32),
            jax.ShapeDtypeStruct((m,), jnp.float32),
            jax.ShapeDtypeStruct((16,), jnp.float32),
        ],
    )(x, w, avec, bvec)


def _tc_mid_body(p_ref, dp_ref, b_ref, w_ref, av_ref, bv_ref,
                 h_ref, as_ref, ad_ref, c_ref):
    den = dp_ref[0] + dp_ref[1] + 1e-16
    t = (p_ref[0] + p_ref[1]) / den[:, None] + b_ref[...][None, :]
    t = jnp.maximum(t, 0.0)
    h = jnp.dot(t, w_ref[...], preferred_element_type=jnp.float32)
    h_ref[...] = h
    asrc = jnp.sum(h * av_ref[...][None, :], axis=1)
    adst = jnp.sum(h * bv_ref[...][None, :], axis=1)
    as_ref[...] = asrc
    ad_ref[...] = adst
    c_ref[...] = jnp.broadcast_to(jnp.max(asrc) + jnp.max(adst), (16,))


def _tc_mid(parts, dp, b, w, avec, bvec):
    m = parts.shape[1]
    d = w.shape[1]
    return pl.pallas_call(
        _tc_mid_body,
        out_shape=[
            jax.ShapeDtypeStruct((m, d), jnp.float32),
            jax.ShapeDtypeStruct((m,), jnp.float32),
            jax.ShapeDtypeStruct((m,), jnp.float32),
            jax.ShapeDtypeStruct((16,), jnp.float32),
        ],
    )(parts, dp, b, w, avec, bvec)


def _tc_z_body(p_ref, dp_ref, b_ref, z_ref):
    den = dp_ref[0] + dp_ref[1] + 1e-16
    z_ref[...] = (p_ref[0] + p_ref[1]) / den[:, None] + b_ref[...][None, :]


def _tc_z(parts, dp, b):
    return pl.pallas_call(
        _tc_z_body,
        out_shape=jax.ShapeDtypeStruct(parts.shape[1:], jnp.float32),
    )(parts, dp, b)


def _tc_fold_body(a_ref, o_ref):
    lanes = lax.broadcasted_iota(jnp.int32, (128, 8), 0)
    cols = lax.broadcasted_iota(jnp.int32, (128, 8), 1)
    s = jnp.where(lanes // 16 == cols, 1.0, 0.0)
    o_ref[...] = jnp.dot(a_ref[...], s, preferred_element_type=jnp.float32)


def _tc_fold(acc):
    return pl.pallas_call(
        _tc_fold_body,
        out_shape=jax.ShapeDtypeStruct((acc.shape[0], 8), jnp.float32),
    )(acc)


def _sc_layer(h, asrc, adst, cvec, src, dst, stage_h, packed=False):
    m = asrc.shape[0]
    gw = h.shape[1]
    d = gw * 2 if packed else gw
    ng = d // 16
    nrows = 10048 if packed else NPAD

    @functools.partial(
        pl.kernel,
        out_type=[
            jax.ShapeDtypeStruct((2, NPAD, d), jnp.float32),
            jax.ShapeDtypeStruct((2 * NPAD,), jnp.float32),
        ],
        mesh=_MESH,
        compiler_params=_SC_PARAMS,
        scratch_types=[
            pltpu.VMEM((NPAD,), jnp.float32),
            pltpu.VMEM((NPAD,), jnp.float32),
            pltpu.VMEM((16,), jnp.float32),
            pltpu.VMEM((4, 128), jnp.int32),
            pltpu.VMEM((4, 128), jnp.int32),
            pltpu.VMEM((512,), jnp.float32),
            pltpu.VMEM((2, 128, gw), jnp.float32),
            pltpu.VMEM((2, 128, d) if packed else (16,),
                       jnp.float32),
            pltpu.VMEM((640,), jnp.float32),
            pltpu.VMEM_SHARED((NPAD, d), jnp.float32),
            pltpu.VMEM_SHARED((NPAD,), jnp.float32),
            pltpu.VMEM_SHARED((nrows, gw) if stage_h else (16,),
                              jnp.float32),
            pltpu.SemaphoreType.DMA,
            pltpu.SemaphoreType.DMA,
            pltpu.SemaphoreType.DMA,
            pltpu.SemaphoreType.DMA,
            pltpu.SemaphoreType.DMA,
            pltpu.SemaphoreType.DMA,
            pltpu.SemaphoreType.DMA,
            pltpu.SemaphoreType.DMA,
            pltpu.SemaphoreType.DMA,
            pltpu.SemaphoreType.DMA,
            pltpu.SemaphoreType.DMA,
            pltpu.SemaphoreType.DMA,
        ],
    )
    def scl(h_h, as_h, ad_h, c_h, src_h, dst_h, outp_h, dp_h,
            asrc_v, adst_v, c_v, src_v, dst_v, w_v, rows_v, rows2_v, zbuf,
            out_sh, den_sh, h_sh,
            ld0, ld1, ld2, ld3, g0, g1, sc0, sc1, w0, w1, w2, w3):
        cid = lax.axis_index("c")
        sid = lax.axis_index("s")
        wid = sid * 2 + cid
        base = wid * T_EDGE
        ld_sems = [ld0, ld1, ld2, ld3]
        g_sems = [g0, g1]
        sc_sems = [sc0, sc1]
        w_sems = [w0, w1, w2, w3]

        srcbuf = rows2_v if packed else rows_v

        @pl.loop(0, 128)
        def _(r):
            for g_ in range(ng):
                srcbuf[0, r, pl.ds(g_ * 16, 16)] = jnp.zeros((16,), jnp.float32)

        @pl.loop(0, 640 // 16)
        def _(i):
            zbuf[pl.ds(i * 16, 16)] = jnp.zeros((16,), jnp.float32)

        pltpu.sync_copy(zbuf, den_sh.at[pl.ds(sid * 640, 640)])

        @pl.loop(0, 5)
        def _(i):
            pltpu.sync_copy(srcbuf.at[0],
                            out_sh.at[pl.ds((sid * 5 + i) * 128, 128)])

        pltpu.sync_copy(as_h.at[pl.ds(0, m)], asrc_v.at[pl.ds(0, m)])
        pltpu.sync_copy(ad_h.at[pl.ds(0, m)], adst_v.at[pl.ds(0, m)])
        pltpu.sync_copy(c_h, c_v)
        if stage_h:
            soff = jnp.minimum(sid * 640, nrows - 640)
            pltpu.sync_copy(h_h.at[pl.ds(soff, 640)],
                            h_sh.at[pl.ds(soff, 640)])
        plsc.subcore_barrier()

        def load_start(b4, c):
            off = pl.multiple_of(
                jnp.minimum(base + c * 128, E_PAD - 128), 128)
            pltpu.make_async_copy(
                src_h.at[pl.ds(off, 128)], src_v.at[b4], ld_sems[b4]).start()
            pltpu.make_async_copy(
                dst_h.at[pl.ds(off, 128)], dst_v.at[b4], ld_sems[b4]).start()

        def load_wait(b4):
            pltpu.make_async_copy(
                src_h.at[pl.ds(0, 128)], src_v.at[b4], ld_sems[b4]).wait()
            pltpu.make_async_copy(
                dst_h.at[pl.ds(0, 128)], dst_v.at[b4], ld_sems[b4]).wait()

        h_src = h_sh if stage_h else h_h

        def gather_start(b4, b2):
            pltpu.make_async_copy(
                h_src.at[src_v.at[b4]], rows_v.at[b2], g_sems[b2]).start()

        def gather_wait(b4, b2):
            pltpu.make_async_copy(
                h_src.at[src_v.at[b4]], rows_v.at[b2], g_sems[b2]).wait()

        def scatter_start(b4, b2):
            pltpu.async_copy(srcbuf.at[b2], out_sh.at[dst_v.at[b4]],
                             sc_sems[b2], add=True)

        def scatter_wait(b4, b2):
            pltpu.make_async_copy(
                srcbuf.at[b2], out_sh.at[dst_v.at[b4]], sc_sems[b2]).wait()

        def compute_w(b4, c):
            cb = base + c * 128
            for k in range(8):
                sv = src_v[b4, pl.ds(k * 16, 16)]
                dv = dst_v[b4, pl.ds(k * 16, 16)]
                av = plsc.load_gather(asrc_v, [sv])
                bv = plsc.load_gather(adst_v, [dv])
                e = av + bv
                e = jnp.where(e >= 0.0, e, 0.2 * e)
                wv = jnp.exp(e - c_v[...])
                gid = lax.iota(jnp.int32, 16) + (cb + k * 16)
                wv = jnp.where(gid < E, wv, 0.0)
                w_v[pl.ds(b4 * 128 + k * 16, 16)] = wv

        def wscatter_start(b4):
            pltpu.async_copy(w_v.at[pl.ds(b4 * 128, 128)],
                             den_sh.at[dst_v.at[b4]], w_sems[b4], add=True)

        def wscatter_wait(b4):
            pltpu.make_async_copy(
                w_v.at[pl.ds(b4 * 128, 128)],
                den_sh.at[dst_v.at[b4]], w_sems[b4]).wait()

        def scale(b4, b2):
            @plsc.parallel_loop(0, 128, unroll=8)
            def _(r):
                widx = jnp.full((16,), b4 * 128, jnp.int32) + r
                wvec = plsc.load_gather(w_v, [widx])
                if packed:
                    for g_ in range(gw // 16):
                        pk = rows_v[b2, r, pl.ds(g_ * 16, 16)]
                        lo, hi = plsc.unpack(
                            plsc.bitcast(pk, jnp.bfloat16),
                            format=plsc.PackFormat.INTERLEAVED)
                        rows2_v[b2, r, pl.ds(g_ * 32, 16)] = lo * wvec
                        rows2_v[b2, r, pl.ds(g_ * 32 + 16, 16)] = hi * wvec
                else:
                    for g_ in range(ng):
                        rows_v[b2, r, pl.ds(g_ * 16, 16)] = (
                            rows_v[b2, r, pl.ds(g_ * 16, 16)] * wvec)

        def half(c, b4, b2, first, wwait):
            gather_wait(b4, b2)
            compute_w(b4, c)
            wscatter_start(b4)
            scale(b4, b2)
            scatter_start(b4, b2)
            if wwait:
                wscatter_wait((b4 + 2) % 4)
            load_start((b4 + 2) % 4, c + 2)
            nb4 = (b4 + 1) % 4
            nb2 = 1 - b2
            load_wait(nb4)
            if not first:
                scatter_wait((b4 + 3) % 4, nb2)
            gather_start(nb4, nb2)

        load_start(0, 0)
        load_start(1, 1)
        load_wait(0)
        gather_start(0, 0)
        half(0, 0, 0, True, False)
        half(1, 1, 1, False, False)
        half(2, 2, 0, False, True)
        half(3, 3, 1, False, True)

        @pl.loop(1, (NCH - 4) // 4 + 1)
        def _(gi):
            c0 = 4 * gi
            half(c0, 0, 0, False, True)
            half(c0 + 1, 1, 1, False, True)
            half(c0 + 2, 2, 0, False, True)
            half(c0 + 3, 3, 1, False, True)

        gather_wait(0, 0)
        load_wait(1)
        scatter_wait(3, 1)
        wscatter_wait(2)
        wscatter_wait(3)
        plsc.subcore_barrier()

        @pl.loop(0, 5)
        def _(i):
            s = (sid * 5 + i) * 128
            pltpu.sync_copy(out_sh.at[pl.ds(s, 128)],
                            outp_h.at[cid, pl.ds(s, 128)])

        pltpu.sync_copy(den_sh.at[pl.ds(sid * 640, 640)],
                        dp_h.at[pl.ds(cid * NPAD + sid * 640, 640)])

    return scl(h, asrc, adst, cvec, src, dst)


def _sc_decode(z, ei0, ei1):
    @functools.partial(
        pl.kernel,
        out_type=jax.ShapeDtypeStruct((P_PAD // 8, 128), jnp.float32),
        mesh=_MESH,
        compiler_params=_SC_PARAMS,
        scratch_types=[
            pltpu.VMEM((2, 128), jnp.int32),
            pltpu.VMEM((2, 128), jnp.int32),
            pltpu.VMEM((2, 128, 32), jnp.float32),
            pltpu.VMEM((2, 128, 32), jnp.float32),
            pltpu.VMEM((2, 16, 128), jnp.float32),
            pltpu.VMEM_SHARED((NPAD, 32), jnp.float32),
            pltpu.SemaphoreType.DMA,
            pltpu.SemaphoreType.DMA,
            pltpu.SemaphoreType.DMA,
            pltpu.SemaphoreType.DMA,
            pltpu.SemaphoreType.DMA,
            pltpu.SemaphoreType.DMA,
        ],
    )
    def scd(z_h, e0_h, e1_h, acc_h,
            i0_v, i1_v, r0_v, r1_v, acc_v, z_sh,
            ld0, ld1, g0, g1, st0, st1):
        cid = lax.axis_index("c")
        sid = lax.axis_index("s")
        wid = sid * 2 + cid
        base = wid * T_DEC
        ld_sems = [ld0, ld1]
        g_sems = [g0, g1]
        st_sems = [st0, st1]

        pltpu.sync_copy(z_h.at[pl.ds(sid * 640, 640)],
                        z_sh.at[pl.ds(sid * 640, 640)])
        plsc.subcore_barrier()

        def load_start(b, c):
            off = pl.multiple_of(
                jnp.minimum(base + c * 128, P_PAD - 128), 128)
            pltpu.make_async_copy(
                e0_h.at[pl.ds(off, 128)], i0_v.at[b], ld_sems[b]).start()
            pltpu.make_async_copy(
                e1_h.at[pl.ds(off, 128)], i1_v.at[b], ld_sems[b]).start()

        def load_wait(b):
            pltpu.make_async_copy(
                e0_h.at[pl.ds(0, 128)], i0_v.at[b], ld_sems[b]).wait()
            pltpu.make_async_copy(
                e1_h.at[pl.ds(0, 128)], i1_v.at[b], ld_sems[b]).wait()

        def gather_start(b):
            pltpu.make_async_copy(
                z_sh.at[i0_v.at[b]], r0_v.at[b], g_sems[b]).start()
            pltpu.make_async_copy(
                z_sh.at[i1_v.at[b]], r1_v.at[b], g_sems[b]).start()

        def gather_wait(b):
            pltpu.make_async_copy(
                z_sh.at[i0_v.at[b]], r0_v.at[b], g_sems[b]).wait()
            pltpu.make_async_copy(
                z_sh.at[i1_v.at[b]], r1_v.at[b], g_sems[b]).wait()

        def store_start(b, c):
            off = pl.multiple_of(base // 8 + c * 16, 16)
            pltpu.make_async_copy(
                acc_v.at[b], acc_h.at[pl.ds(off, 16)], st_sems[b]).start()

        def store_wait(b):
            pltpu.make_async_copy(
                acc_v.at[b], acc_h.at[pl.ds(0, 16)], st_sems[b]).wait()

        def half(c, b, first):
            gather_wait(b)
            if not first:
                store_wait(b)

            @plsc.parallel_loop(0, 128, unroll=8)
            def _(r):
                p0 = r0_v[b, r, pl.ds(0, 16)] * r1_v[b, r, pl.ds(0, 16)]
                p1 = r0_v[b, r, pl.ds(16, 16)] * r1_v[b, r, pl.ds(16, 16)]
                acc_v[b, r // 8, pl.ds((r % 8) * 16, 16)] = p0 + p1

            store_start(b, c)
            load_start(b, c + 2)
            nb = 1 - b
            load_wait(nb)
            gather_start(nb)

        load_start(0, 0)
        load_start(1, 1)
        load_wait(0)
        gather_start(0)
        half(0, 0, True)
        half(1, 1, True)

        @pl.loop(1, NCH_D // 2)
        def _(gi):
            half(2 * gi, 0, False)
            half(2 * gi + 1, 1, False)

        gather_wait(0)
        load_wait(1)
        store_wait(0)
        store_wait(1)

    return scd(z, ei0, ei1)


def kernel(x, edge_index, pos_edge_index, neg_edge_index,
           W1, a1_src, a1_dst, b1, W2, a2_src, a2_dst, b2):
    loops = jnp.arange(N, dtype=edge_index.dtype)
    ei = jnp.concatenate(
        [edge_index, jnp.stack([loops, loops], axis=0)], axis=1)
    pad = E_PAD - E
    src = jnp.pad(ei[0], (0, pad))
    dst = jnp.pad(ei[1], (0, pad))

    dec = jnp.concatenate([pos_edge_index, neg_edge_index], axis=1)
    dpad = P_PAD - P
    d0 = jnp.pad(dec[0], (0, dpad))
    d1 = jnp.pad(dec[1], (0, dpad))

    h1, as1, ad1, c1 = _tc_embed(x, W1, a1_src, a1_dst)
    dh = h1.shape[1]
    h1b = jnp.pad(h1, ((0, NPAD - h1.shape[0]), (0, 0))).astype(jnp.bfloat16)
    hp1 = jax.lax.bitcast_convert_type(
        h1b.reshape(NPAD, dh // 2, 2), jnp.float32)
    idx = jnp.arange(dh)
    g_, r_ = idx // 32, idx % 32
    porder = 32 * g_ + jnp.where(r_ < 16, 2 * r_, 2 * (r_ - 16) + 1)
    outp1, dp1 = _sc_layer(hp1, as1, ad1, c1, src, dst,
                           stage_h=True, packed=True)

    h2, as2, ad2, c2 = _tc_mid(outp1, dp1.reshape(2, NPAD), b1[porder],
                               W2[porder, :], a2_src, a2_dst)
    outp2, dp2 = _sc_layer(h2, as2, ad2, c2, src, dst, stage_h=True)

    z = _tc_z(outp2, dp2.reshape(2, NPAD), b2)
    acc = _sc_decode(z, d0, d1)
    logits_pad = _tc_fold(acc)
    return logits_pad.reshape(P_PAD)[:P]
